# baseline (speedup 1.0000x reference)
import jax
import jax.numpy as jnp
from jax.experimental import pallas as pl

EMB = 8
LSZ = 4


def _vq_body(x_ref, wm2_ref, wt_ref, wn_ref, q_ref, p_ref, cb_ref):
    wm2 = wm2_ref[...]
    wt = wt_ref[...]
    wn = wn_ref[...]
    for l in range(LSZ):
        x = x_ref[l]
        dots = jax.lax.dot(wm2, x, preferred_element_type=jnp.float32)
        dist = dots + wn
        g = dist
        for k in (1, 2, 4):
            g = jnp.minimum(g, jnp.roll(g, k, axis=0))
        onehot = (dist == g).astype(jnp.float32)
        q = jax.lax.dot(wt, onehot, preferred_element_type=jnp.float32)
        q_ref[l] = q
        p_ref[l] = q
    for e in range(EMB):
        cb_ref[e] = jnp.broadcast_to(wt[:, e:e + 1], cb_ref.shape[1:])


def kernel(latent, W):
    B = latent.shape[0]
    xt = latent.transpose(1, 2, 0)
    wm2 = (-2.0) * W
    wt = W.T
    wn = jnp.sum(W * W, axis=1, keepdims=True)

    C = 16384
    grid = (B // C,)
    qt, pt, cbt = pl.pallas_call(
        _vq_body,
        grid=grid,
        in_specs=[
            pl.BlockSpec((LSZ, EMB, C), lambda i: (0, 0, i)),
            pl.BlockSpec((EMB, EMB), lambda i: (0, 0)),
            pl.BlockSpec((EMB, EMB), lambda i: (0, 0)),
            pl.BlockSpec((EMB, 1), lambda i: (0, 0)),
        ],
        out_specs=[
            pl.BlockSpec((LSZ, EMB, C), lambda i: (0, 0, i)),
            pl.BlockSpec((LSZ, EMB, C), lambda i: (0, 0, i)),
            pl.BlockSpec((EMB, EMB, C), lambda i: (0, 0, i)),
        ],
        out_shape=[
            jax.ShapeDtypeStruct((LSZ, EMB, B), jnp.float32),
            jax.ShapeDtypeStruct((LSZ, EMB, B), jnp.float32),
            jax.ShapeDtypeStruct((EMB, EMB, B), jnp.float32),
        ],
    )(xt, wm2, wt, wn)

    q = qt.transpose(2, 0, 1)
    p = pt.transpose(2, 0, 1)
    cb = cbt.transpose(2, 0, 1)
    return (p, q, cb)
